# baseline (speedup 1.0000x reference)
import functools

import jax
import jax.numpy as jnp
from jax.experimental import pallas as pl
from jax.experimental.pallas import tpu as pltpu

_NUM_EXPERTS = 8
_BT = 1024
_NBUF = 8


def _router_kernel(x_hbm, wt_ref, gates_ref, idx_ref, loss_ref,
                   buf_ref, acc_ref, sem, *, nblocks, ntokens):
    def _copy(block, slot):
        return pltpu.make_async_copy(
            x_hbm.at[pl.ds(block * _BT, _BT), :],
            buf_ref.at[slot],
            sem.at[slot],
        )

    for k in range(_NBUF):
        _copy(k, k).start()

    acc_ref[...] = jnp.zeros_like(acc_ref)

    def body(i, carry):
        slot = jax.lax.rem(i, _NBUF)
        _copy(i, slot).wait()
        x = buf_ref[slot]
        logits = jnp.dot(x, wt_ref[...],
                         preferred_element_type=jnp.float32)
        lt = logits.T
        e = jax.lax.broadcasted_iota(jnp.int32, (_NUM_EXPERTS, _BT), 0)

        m1 = jnp.max(lt, axis=0, keepdims=True)
        i1 = jnp.min(jnp.where(lt == m1, e, _NUM_EXPERTS), axis=0,
                     keepdims=True)
        masked = jnp.where(e == i1, -jnp.inf, lt)
        m2 = jnp.max(masked, axis=0, keepdims=True)
        i2 = jnp.min(jnp.where(masked == m2, e, _NUM_EXPERTS), axis=0,
                     keepdims=True)

        ed = jnp.exp(m2 - m1)
        g2 = ed / (1.0 + ed)
        g1 = 1.0 - g2

        gt = jnp.where(e == i1, g1, jnp.where(e == i2, g2, jnp.float32(0.0)))
        gates_ref[pl.ds(i * _BT, _BT), :] = gt.T
        idx_ref[pl.ds(i * _BT, _BT), :] = jnp.concatenate([i1, i2], axis=0).T
        acc_ref[...] += jnp.sum(gt, axis=1, keepdims=True)

        nxt = i + _NBUF

        @pl.when(nxt < nblocks)
        def _prefetch():
            _copy(nxt, slot).start()

        return carry

    jax.lax.fori_loop(0, nblocks, body, 0)

    usage = acc_ref[...] / jnp.float32(ntokens)
    log_usage = jnp.maximum(jnp.log(usage), -1e9)
    u = jnp.float32(1.0 / _NUM_EXPERTS)
    loss_ref[...] = jnp.sum(u * (jnp.log(u) - log_usage)).reshape(1, 1)


def kernel(input_tensor, W):
    B, S, D = input_tensor.shape
    E = W.shape[0]
    n = B * S
    x = input_tensor.reshape(n, D)
    wt = W.T
    nblocks = n // _BT

    gates, idx, loss = pl.pallas_call(
        functools.partial(_router_kernel, nblocks=nblocks, ntokens=n),
        in_specs=[
            pl.BlockSpec(memory_space=pl.ANY),
            pl.BlockSpec(memory_space=pltpu.VMEM),
        ],
        out_specs=[
            pl.BlockSpec(memory_space=pltpu.VMEM),
            pl.BlockSpec(memory_space=pltpu.VMEM),
            pl.BlockSpec(memory_space=pltpu.VMEM),
        ],
        out_shape=[
            jax.ShapeDtypeStruct((n, E), jnp.float32),
            jax.ShapeDtypeStruct((n, 2), jnp.int32),
            jax.ShapeDtypeStruct((1, 1), jnp.float32),
        ],
        scratch_shapes=[
            pltpu.VMEM((_NBUF, _BT, D), jnp.float32),
            pltpu.VMEM((E, 1), jnp.float32),
            pltpu.SemaphoreType.DMA((_NBUF,)),
        ],
    )(x, wt)

    return (gates.reshape(B, S, E), idx.reshape(B, S, 2), loss.reshape(()))
